# baseline (speedup 1.0000x reference)
import jax
import jax.numpy as jnp
from jax.experimental import pallas as pl

_V_TILE = 3


def _head_body(xt_ref, xf_ref, wt_ref, wf_ref, b_ref, o_ref):
    vt, P, B, D = xt_ref.shape
    tw = o_ref.shape[1]
    dn = (((1,), (0,)), ((), ()))
    acc = b_ref[...]
    for p in range(P):
        xt = xt_ref[:, p, :, :].reshape(vt * B, D)
        xf = xf_ref[:, p, :, :].reshape(vt * B, D)
        wt = wt_ref[p, :, :]
        wf = wf_ref[p, :, :]
        acc += jax.lax.dot_general(xt, wt, dn, preferred_element_type=jnp.float32)
        acc += jax.lax.dot_general(xf, wf, dn, preferred_element_type=jnp.float32)
    o_ref[...] = acc


def kernel(x_time, x_frequency, W, b):
    B, V, D, P = x_time.shape
    K = D * P
    TW = W.shape[0]

    xt = jnp.transpose(x_time, (1, 3, 0, 2))
    xf = jnp.transpose(x_frequency, (1, 3, 0, 2))

    Wt = W[:, :K].reshape(TW, D, P).transpose(2, 1, 0)
    Wf = W[:, K:].reshape(TW, D, P).transpose(2, 1, 0)
    b2 = b.reshape(1, TW)

    grid = (V // _V_TILE,)
    out = pl.pallas_call(
        _head_body,
        grid=grid,
        in_specs=[
            pl.BlockSpec((_V_TILE, P, B, D), lambda i: (i, 0, 0, 0)),
            pl.BlockSpec((_V_TILE, P, B, D), lambda i: (i, 0, 0, 0)),
            pl.BlockSpec((P, D, TW), lambda i: (0, 0, 0)),
            pl.BlockSpec((P, D, TW), lambda i: (0, 0, 0)),
            pl.BlockSpec((1, TW), lambda i: (0, 0)),
        ],
        out_specs=pl.BlockSpec((_V_TILE * B, TW), lambda i: (i, 0)),
        out_shape=jax.ShapeDtypeStruct((V * B, TW), jnp.float32),
    )(xt, xf, Wt, Wf, b2)

    return out.reshape(V, B, TW).transpose(1, 0, 2)


# speedup vs baseline: 1.3663x; 1.3663x over previous
import jax
import jax.numpy as jnp
from jax.experimental import pallas as pl

_V_TILE = 107
_P_CHUNK = 2


def _head_body(xt_ref, xf_ref, wt_ref, wf_ref, b_ref, o_ref):
    p = pl.program_id(1)
    vt, pc, bb, d = xt_ref.shape
    mb = vt * bb
    tw = o_ref.shape[1]

    dn = (((1,), (0,)), ((), ()))
    acc = jax.lax.dot_general(
        xt_ref[:, 0, :, :].reshape(mb, d), wt_ref[0], dn,
        preferred_element_type=jnp.float32)
    acc += jax.lax.dot_general(
        xf_ref[:, 0, :, :].reshape(mb, d), wf_ref[0], dn,
        preferred_element_type=jnp.float32)
    for q in range(1, pc):
        acc += jax.lax.dot_general(
            xt_ref[:, q, :, :].reshape(mb, d), wt_ref[q], dn,
            preferred_element_type=jnp.float32)
        acc += jax.lax.dot_general(
            xf_ref[:, q, :, :].reshape(mb, d), wf_ref[q], dn,
            preferred_element_type=jnp.float32)

    @pl.when(p == 0)
    def _init():
        o_ref[...] = acc + b_ref[...]

    @pl.when(p != 0)
    def _accum():
        o_ref[...] += acc


def kernel(x_time, x_frequency, W, b):
    B, V, D, P = x_time.shape
    K = D * P
    TW = W.shape[0]

    xt = jnp.transpose(x_time, (1, 3, 0, 2))
    xf = jnp.transpose(x_frequency, (1, 3, 0, 2))

    Wt = W[:, :K].reshape(TW, D, P).transpose(2, 1, 0)
    Wf = W[:, K:].reshape(TW, D, P).transpose(2, 1, 0)
    b2 = b.reshape(1, TW)

    grid = (V // _V_TILE, P // _P_CHUNK)
    out = pl.pallas_call(
        _head_body,
        grid=grid,
        in_specs=[
            pl.BlockSpec((_V_TILE, _P_CHUNK, B, D), lambda i, p: (i, p, 0, 0)),
            pl.BlockSpec((_V_TILE, _P_CHUNK, B, D), lambda i, p: (i, p, 0, 0)),
            pl.BlockSpec((_P_CHUNK, D, TW), lambda i, p: (p, 0, 0)),
            pl.BlockSpec((_P_CHUNK, D, TW), lambda i, p: (p, 0, 0)),
            pl.BlockSpec((1, TW), lambda i, p: (0, 0)),
        ],
        out_specs=pl.BlockSpec((_V_TILE * B, TW), lambda i, p: (i, 0)),
        out_shape=jax.ShapeDtypeStruct((V * B, TW), jnp.float32),
    )(xt, xf, Wt, Wf, b2)

    return out.reshape(V, B, TW).transpose(1, 0, 2)
